# baseline (speedup 1.0000x reference)
import jax
import jax.numpy as jnp
from jax.experimental import pallas as pl
from jax.experimental.pallas import tpu as pltpu

_B, _P, _K = 32, 512, 20


def _graph_kernel(shift_ref, pos_ref, W1a_ref, b1a_ref, W1b_ref, b1b_ref,
                  W2_ref, b2_ref, Wh_ref, bh_ref, out_ref):
    f32 = jnp.float32
    iota_q = jax.lax.broadcasted_iota(jnp.int32, (_P, _P), 0).astype(f32)

    def dot(a, b):
        return jax.lax.dot_general(a, b, (((1,), (0,)), ((), ())),
                                   preferred_element_type=f32)

    def pairwise_d2(feat):
        sq = jnp.sum(feat * feat, axis=1, keepdims=True)
        g = jax.lax.dot_general(feat, feat, (((1,), (1,)), ((), ())),
                                preferred_element_type=f32)
        return sq + sq.reshape(1, _P) - 2.0 * g

    def knn_max_multi(d2s, tables, msg_fns, out_dim):
        n = len(d2s)
        d2cs = list(d2s)
        accs = [jnp.full((_P, out_dim), -jnp.inf, dtype=f32)] * n
        for _ in range(_K):
            for i in range(n):
                d2c = d2cs[i]
                m = jnp.min(d2c, axis=0, keepdims=True)
                am = jnp.min(jnp.where(d2c == m, iota_q, float(_P)), axis=0,
                             keepdims=True)
                onehot_b = iota_q == am
                onehot = onehot_b.astype(f32)
                gathered = jax.lax.dot_general(
                    onehot, tables[i], (((0,), (0,)), ((), ())),
                    preferred_element_type=f32)
                accs[i] = jnp.maximum(accs[i], msg_fns[i](gathered))
                d2cs[i] = jnp.where(onehot_b, jnp.inf, d2c)
        return accs

    ngr = pos_ref.shape[0]
    xs = [pos_ref[i] + shift_ref[0, 0] for i in range(ngr)]

    W1a_top = W1a_ref[0:3, :]
    W1a_bot = W1a_ref[3:6, :]
    c1s = [dot(x, W1a_top - W1a_bot) + b1a_ref[0] for x in xs]

    def mk_msg1(c1):
        return lambda xj: dot(jax.nn.relu(c1 + dot(xj, W1a_bot)),
                              W1b_ref[...])

    f1s = knn_max_multi([pairwise_d2(x) for x in xs], xs,
                        [mk_msg1(c1) for c1 in c1s], 64)
    f1s = [f1 + b1b_ref[0] for f1 in f1s]

    W2_top = W2_ref[0:64, :]
    W2_bot = W2_ref[64:128, :]
    c2s = [dot(f1, W2_top - W2_bot) + b2_ref[0] for f1 in f1s]
    msg2 = lambda fj: dot(fj, W2_bot)

    f2s = knn_max_multi([pairwise_d2(f1) for f1 in f1s], f1s,
                        [msg2] * ngr, 128)

    for i in range(ngr):
        h = (dot(f1s[i], Wh_ref[0:64, :])
             + dot(c2s[i] + f2s[i], Wh_ref[64:192, :]) + bh_ref[0])
        out_ref[i] = jnp.max(h, axis=0, keepdims=True)


def kernel(pos, batch, W1a, b1a, W1b, b1b, W2, b2, Wh, bh):
    nb = _B
    pp = pos.shape[0] // nb
    shift = (batch[-1].astype(jnp.int32) + 1 - nb).astype(pos.dtype)
    posb = pos.reshape(nb, pp, 3)
    shift2d = shift.reshape(1, 1)

    full = lambda shape: pl.BlockSpec(shape, lambda g: (0,) * len(shape))
    gpb = 2
    out = pl.pallas_call(
        _graph_kernel,
        grid=(nb // gpb,),
        in_specs=[
            full((1, 1)),
            pl.BlockSpec((gpb, pp, 3), lambda g: (g, 0, 0)),
            full((6, 64)), full((1, 64)),
            full((64, 64)), full((1, 64)),
            full((128, 128)), full((1, 128)),
            full((192, 128)), full((1, 128)),
        ],
        out_specs=pl.BlockSpec((gpb, 1, 128), lambda g: (g, 0, 0)),
        out_shape=jax.ShapeDtypeStruct((nb, 1, 128), jnp.float32),
        compiler_params=pltpu.CompilerParams(
            dimension_semantics=("parallel",)),
    )(shift2d, posb, W1a, b1a.reshape(1, 64), W1b, b1b.reshape(1, 64),
      W2, b2.reshape(1, 128), Wh, bh.reshape(1, 128))
    return out.reshape(nb, 128)


# speedup vs baseline: 1.5883x; 1.5883x over previous
import jax
import jax.numpy as jnp
from jax.experimental import pallas as pl
from jax.experimental.pallas import tpu as pltpu

_B, _P, _K = 32, 512, 20


def _graph_kernel(shift_ref, pos_ref, W1a_ref, b1a_ref, W1b_ref, b1b_ref,
                  W2_ref, b2_ref, Wh_ref, bh_ref, out_ref):
    f32 = jnp.float32
    iota_q = jax.lax.broadcasted_iota(jnp.int32, (_P, _P), 1).astype(f32)

    def dot(a, b):
        return jax.lax.dot_general(a, b, (((1,), (0,)), ((), ())),
                                   preferred_element_type=f32)

    def pairwise_d2(feat):
        sq = jnp.sum(feat * feat, axis=1, keepdims=True)
        g = jax.lax.dot_general(feat, feat, (((1,), (1,)), ((), ())),
                                preferred_element_type=f32)
        return sq + sq.reshape(1, _P) - 2.0 * g

    def knn_max_multi(d2s, tables, msg_fns, out_dim):
        n = len(d2s)
        d2cs = list(d2s)
        accs = [jnp.full((_P, out_dim), -jnp.inf, dtype=f32)] * n
        for _ in range(_K):
            for i in range(n):
                d2c = d2cs[i]
                m = jnp.min(d2c, axis=1, keepdims=True)
                am = jnp.min(jnp.where(d2c == m, iota_q, float(_P)), axis=1,
                             keepdims=True)
                onehot_b = iota_q == am
                onehot = onehot_b.astype(f32)
                gathered = dot(onehot, tables[i])
                accs[i] = jnp.maximum(accs[i], msg_fns[i](gathered))
                d2cs[i] = jnp.where(onehot_b, jnp.inf, d2c)
        return accs

    ngr = pos_ref.shape[0]
    xs = [pos_ref[i] + shift_ref[0, 0] for i in range(ngr)]

    W1a_top = W1a_ref[0:3, :]
    W1a_bot = W1a_ref[3:6, :]
    c1s = [dot(x, W1a_top - W1a_bot) + b1a_ref[0] for x in xs]

    def mk_msg1(c1):
        return lambda xj: dot(jax.nn.relu(c1 + dot(xj, W1a_bot)),
                              W1b_ref[...])

    f1s = knn_max_multi([pairwise_d2(x) for x in xs], xs,
                        [mk_msg1(c1) for c1 in c1s], 64)
    f1s = [f1 + b1b_ref[0] for f1 in f1s]

    W2_top = W2_ref[0:64, :]
    W2_bot = W2_ref[64:128, :]
    c2s = [dot(f1, W2_top - W2_bot) + b2_ref[0] for f1 in f1s]
    msg2 = lambda fj: dot(fj, W2_bot)

    f2s = knn_max_multi([pairwise_d2(f1) for f1 in f1s], f1s,
                        [msg2] * ngr, 128)

    for i in range(ngr):
        h = (dot(f1s[i], Wh_ref[0:64, :])
             + dot(c2s[i] + f2s[i], Wh_ref[64:192, :]) + bh_ref[0])
        out_ref[i] = jnp.max(h, axis=0, keepdims=True)


def kernel(pos, batch, W1a, b1a, W1b, b1b, W2, b2, Wh, bh):
    nb = _B
    pp = pos.shape[0] // nb
    shift = (batch[-1].astype(jnp.int32) + 1 - nb).astype(pos.dtype)
    posb = pos.reshape(nb, pp, 3)
    shift2d = shift.reshape(1, 1)

    full = lambda shape: pl.BlockSpec(shape, lambda g: (0,) * len(shape))
    gpb = 2
    out = pl.pallas_call(
        _graph_kernel,
        grid=(nb // gpb,),
        in_specs=[
            full((1, 1)),
            pl.BlockSpec((gpb, pp, 3), lambda g: (g, 0, 0)),
            full((6, 64)), full((1, 64)),
            full((64, 64)), full((1, 64)),
            full((128, 128)), full((1, 128)),
            full((192, 128)), full((1, 128)),
        ],
        out_specs=pl.BlockSpec((gpb, 1, 128), lambda g: (g, 0, 0)),
        out_shape=jax.ShapeDtypeStruct((nb, 1, 128), jnp.float32),
        compiler_params=pltpu.CompilerParams(
            dimension_semantics=("parallel",)),
    )(shift2d, posb, W1a, b1a.reshape(1, 64), W1b, b1b.reshape(1, 64),
      W2, b2.reshape(1, 128), Wh, bh.reshape(1, 128))
    return out.reshape(nb, 128)
